# baseline (speedup 1.0000x reference)
import functools

import jax
import jax.numpy as jnp
from jax import lax
from jax.experimental import pallas as pl
from jax.experimental.pallas import tpu as pltpu
from jax.experimental.pallas import tpu_sc as plsc

N_ROWS = 320000
N_SEG = 10000
N_HID = 128

NC = 2
NS = 16
NW = NC * NS
ROWS_PER_W = N_ROWS // NW
CHUNK = 80
N_CHUNKS = ROWS_PER_W // CHUNK
NBUF = 4
FULL_ROUNDS = 30
SEG_PER_TILE = 640
N_STAGE = 8


def _sc_segment_sum(x, batch3, zeros):
    mesh = plsc.VectorSubcoreMesh(
        core_axis_name="c", subcore_axis_name="s",
        num_cores=NC, num_subcores=NS)

    @functools.partial(
        pl.kernel,
        out_type=jax.ShapeDtypeStruct((NC * N_SEG, N_HID), jnp.float32),
        mesh=mesh,
        compiler_params=pltpu.CompilerParams(use_tc_tiling_on_sc=False),
        scratch_types=dict(
            pooled=pltpu.VMEM_SHARED((N_SEG, N_HID), jnp.float32),
            idx_all=pltpu.VMEM((N_CHUNKS, CHUNK), jnp.int32),
            rows=pltpu.VMEM((NBUF, CHUNK, N_HID), jnp.float32),
            isem=pltpu.SemaphoreType.DMA,
            zsem=pltpu.SemaphoreType.DMA,
            lsem=pltpu.SemaphoreType.DMA((NBUF,)),
            ssem=pltpu.SemaphoreType.DMA((NBUF,)),
        ),
    )
    def body(x_hbm, batch3_hbm, zeros_hbm, out_hbm, *, pooled, idx_all, rows,
             isem, zsem, lsem, ssem):
        c = lax.axis_index("c")
        s = lax.axis_index("s")
        wid = c * NS + s
        row0 = wid * ROWS_PER_W

        def load_desc(i, b):
            return pltpu.make_async_copy(
                x_hbm.at[pl.ds(row0 + i * CHUNK, CHUNK)], rows.at[b],
                lsem.at[b])

        def scat_desc(i, b):
            return pltpu.make_async_copy(
                rows.at[b], pooled.at[idx_all.at[i]], ssem.at[b])

        idx_desc = pltpu.make_async_copy(batch3_hbm.at[wid], idx_all, isem)
        idx_desc.start()
        for b in range(1, NBUF):
            load_desc(b, b).start()
        pltpu.make_async_copy(zeros_hbm, rows.at[0], zsem).start()

        nseg_t = jnp.where(s == NS - 1,
                           N_SEG - (NS - 1) * SEG_PER_TILE, SEG_PER_TILE)
        pltpu.make_async_copy(zeros_hbm, rows.at[0], zsem).wait()

        def zero_desc(k):
            seg0 = s * SEG_PER_TILE + k * CHUNK
            return pltpu.make_async_copy(
                rows.at[0], pooled.at[pl.ds(seg0, CHUNK)], ssem.at[k % NBUF])

        for k in range(N_STAGE):

            @pl.when(k * CHUNK < nseg_t)
            def _():
                if k >= NBUF:
                    zero_desc(k - NBUF).wait()
                zero_desc(k).start()
        for k in range(N_STAGE):
            cond = jnp.logical_and(k * CHUNK < nseg_t,
                                   (k + NBUF) * CHUNK >= nseg_t)

            @pl.when(cond)
            def _():
                zero_desc(k).wait()

        load_desc(0, 0).start()
        idx_desc.wait()
        plsc.subcore_barrier()

        def full_round(g, _):
            for b in range(NBUF):
                i = g * NBUF + b
                load_desc(i, b).wait()
                scat_desc(i, b).start(add=True)
            for b in range(NBUF):
                i = g * NBUF + b
                scat_desc(i, b).wait()

                @pl.when(i + NBUF < N_CHUNKS)
                def _():
                    load_desc(i + NBUF, b).start()
            return 0

        lax.fori_loop(0, FULL_ROUNDS + 1, full_round, 0)

        for b in range(N_CHUNKS - (FULL_ROUNDS + 1) * NBUF):
            i = (FULL_ROUNDS + 1) * NBUF + b
            load_desc(i, b).wait()
            scat_desc(i, b).start(add=True)
            scat_desc(i, b).wait()
        plsc.subcore_barrier()

        def rd_desc(k, p):
            seg0 = s * SEG_PER_TILE + k * CHUNK
            return pltpu.make_async_copy(
                pooled.at[pl.ds(seg0, CHUNK)], rows.at[p], lsem.at[p])

        def wr_desc(k, p):
            seg0 = s * SEG_PER_TILE + k * CHUNK
            return pltpu.make_async_copy(
                rows.at[p], out_hbm.at[pl.ds(c * N_SEG + seg0, CHUNK)],
                ssem.at[p])

        def drain_pair(m, _):
            for p in range(2):
                k = 2 * m + p

                @pl.when(k * CHUNK < nseg_t)
                def _():
                    @pl.when(m >= 1)
                    def _():
                        wr_desc(k - 2, p).wait()
                    rd_desc(k, p).start()
                    rd_desc(k, p).wait()
                    wr_desc(k, p).start()
            return 0

        lax.fori_loop(0, N_STAGE // 2, drain_pair, 0)
        def drain_wait(m, _):
            for p in range(2):
                k = 2 * m + p
                cond = jnp.logical_and(k * CHUNK < nseg_t,
                                       (k + 2) * CHUNK >= nseg_t)

                @pl.when(cond)
                def _():
                    wr_desc(k, p).wait()
            return 0

        lax.fori_loop(0, N_STAGE // 2, drain_wait, 0)

    return body(x, batch3, zeros)


ROW_BLK = 2000


def _mm_body(p0_ref, p1_ref, w_ref, b_ref, o_ref):
    pooled = p0_ref[...] + p1_ref[...]
    o_ref[...] = lax.dot_general(
        pooled, w_ref[...], (((1,), (1,)), ((), ())),
        preferred_element_type=jnp.float32) + b_ref[...]


def _tc_linear(partials, W, b):
    grid = N_SEG // ROW_BLK
    return pl.pallas_call(
        _mm_body,
        grid=(grid,),
        in_specs=[
            pl.BlockSpec((ROW_BLK, N_HID), lambda i: (i, 0)),
            pl.BlockSpec((ROW_BLK, N_HID), lambda i, g=grid: (i + g, 0)),
            pl.BlockSpec((N_HID, N_HID), lambda i: (0, 0)),
            pl.BlockSpec((1, N_HID), lambda i: (0, 0)),
        ],
        out_specs=pl.BlockSpec((ROW_BLK, N_HID), lambda i: (i, 0)),
        out_shape=jax.ShapeDtypeStruct((N_SEG, N_HID), jnp.float32),
    )(partials, partials, W, b.reshape(1, N_HID))


def kernel(x, batch, W, b):
    batch3 = batch.astype(jnp.int32).reshape(NW, N_CHUNKS, CHUNK)
    zeros = jnp.zeros((CHUNK, N_HID), jnp.float32)
    partials = _sc_segment_sum(x, batch3, zeros)
    return _tc_linear(partials, W, b)


# speedup vs baseline: 1.0248x; 1.0248x over previous
import functools

import jax
import jax.numpy as jnp
from jax import lax
from jax.experimental import pallas as pl
from jax.experimental.pallas import tpu as pltpu
from jax.experimental.pallas import tpu_sc as plsc

N_ROWS = 320000
N_SEG = 10000
N_HID = 128

NC = 2
NS = 16
NW = NC * NS
ROWS_PER_W = N_ROWS // NW
CHUNK = 80
N_CHUNKS = ROWS_PER_W // CHUNK
NBUF = 4
FULL_ROUNDS = 30
SEG_PER_TILE = 640
N_STAGE = 8


def _sc_segment_sum(x, batch3):
    mesh = plsc.VectorSubcoreMesh(
        core_axis_name="c", subcore_axis_name="s",
        num_cores=NC, num_subcores=NS)

    @functools.partial(
        pl.kernel,
        out_type=jax.ShapeDtypeStruct((NC * N_SEG, N_HID), jnp.float32),
        mesh=mesh,
        compiler_params=pltpu.CompilerParams(use_tc_tiling_on_sc=False),
        scratch_types=dict(
            pooled=pltpu.VMEM_SHARED((N_SEG, N_HID), jnp.float32),
            idx_all=pltpu.VMEM((N_CHUNKS, CHUNK), jnp.int32),
            rows=pltpu.VMEM((NBUF, CHUNK, N_HID), jnp.float32),
            isem=pltpu.SemaphoreType.DMA,
            lsem=pltpu.SemaphoreType.DMA((NBUF,)),
            ssem=pltpu.SemaphoreType.DMA((NBUF,)),
        ),
    )
    def body(x_hbm, batch3_hbm, out_hbm, *, pooled, idx_all, rows,
             isem, lsem, ssem):
        c = lax.axis_index("c")
        s = lax.axis_index("s")
        wid = c * NS + s
        row0 = wid * ROWS_PER_W

        def load_desc(i, b):
            return pltpu.make_async_copy(
                x_hbm.at[pl.ds(row0 + i * CHUNK, CHUNK)], rows.at[b],
                lsem.at[b])

        def scat_desc(i, b):
            return pltpu.make_async_copy(
                rows.at[b], pooled.at[idx_all.at[i]], ssem.at[b])

        idx_desc = pltpu.make_async_copy(batch3_hbm.at[wid], idx_all, isem)
        idx_desc.start()
        for b in range(1, NBUF):
            load_desc(b, b).start()

        zvec = jnp.zeros((16,), jnp.float32)

        def zrow(i, _):
            for j in range(N_HID // 16):
                rows[0, i, pl.ds(j * 16, 16)] = zvec
            return 0

        lax.fori_loop(0, CHUNK, zrow, 0)
        nseg_t = jnp.where(s == NS - 1,
                           N_SEG - (NS - 1) * SEG_PER_TILE, SEG_PER_TILE)
        for k in range(N_STAGE):
            seg0 = s * SEG_PER_TILE + k * CHUNK

            @pl.when(k * CHUNK < nseg_t)
            def _():
                pltpu.sync_copy(rows.at[0], pooled.at[pl.ds(seg0, CHUNK)])

        load_desc(0, 0).start()
        idx_desc.wait()
        plsc.subcore_barrier()

        def full_round(g, _):
            for b in range(NBUF):
                i = g * NBUF + b
                load_desc(i, b).wait()
                scat_desc(i, b).start(add=True)
            for b in range(NBUF):
                i = g * NBUF + b
                scat_desc(i, b).wait()

                @pl.when(i + NBUF < N_CHUNKS)
                def _():
                    load_desc(i + NBUF, b).start()
            return 0

        lax.fori_loop(0, FULL_ROUNDS + 1, full_round, 0)

        for b in range(N_CHUNKS - (FULL_ROUNDS + 1) * NBUF):
            i = (FULL_ROUNDS + 1) * NBUF + b
            load_desc(i, b).wait()
            scat_desc(i, b).start(add=True)
            scat_desc(i, b).wait()
        plsc.subcore_barrier()

        def rd_desc(k, p):
            seg0 = s * SEG_PER_TILE + k * CHUNK
            return pltpu.make_async_copy(
                pooled.at[pl.ds(seg0, CHUNK)], rows.at[p], lsem.at[p])

        def wr_desc(k, p):
            seg0 = s * SEG_PER_TILE + k * CHUNK
            return pltpu.make_async_copy(
                rows.at[p], out_hbm.at[pl.ds(c * N_SEG + seg0, CHUNK)],
                ssem.at[p])

        def drain_pair(m, _):
            for p in range(2):
                k = 2 * m + p

                @pl.when(k * CHUNK < nseg_t)
                def _():
                    @pl.when(m >= 1)
                    def _():
                        wr_desc(k - 2, p).wait()
                    rd_desc(k, p).start()
                    rd_desc(k, p).wait()
                    wr_desc(k, p).start()
            return 0

        lax.fori_loop(0, N_STAGE // 2, drain_pair, 0)
        def drain_wait(m, _):
            for p in range(2):
                k = 2 * m + p
                cond = jnp.logical_and(k * CHUNK < nseg_t,
                                       (k + 2) * CHUNK >= nseg_t)

                @pl.when(cond)
                def _():
                    wr_desc(k, p).wait()
            return 0

        lax.fori_loop(0, N_STAGE // 2, drain_wait, 0)

    return body(x, batch3)


ROW_BLK = 2000


def _mm_body(p0_ref, p1_ref, w_ref, b_ref, o_ref):
    pooled = p0_ref[...] + p1_ref[...]
    o_ref[...] = lax.dot_general(
        pooled, w_ref[...], (((1,), (1,)), ((), ())),
        preferred_element_type=jnp.float32) + b_ref[...]


def _tc_linear(partials, W, b):
    grid = N_SEG // ROW_BLK
    return pl.pallas_call(
        _mm_body,
        grid=(grid,),
        in_specs=[
            pl.BlockSpec((ROW_BLK, N_HID), lambda i: (i, 0)),
            pl.BlockSpec((ROW_BLK, N_HID), lambda i, g=grid: (i + g, 0)),
            pl.BlockSpec((N_HID, N_HID), lambda i: (0, 0)),
            pl.BlockSpec((1, N_HID), lambda i: (0, 0)),
        ],
        out_specs=pl.BlockSpec((ROW_BLK, N_HID), lambda i: (i, 0)),
        out_shape=jax.ShapeDtypeStruct((N_SEG, N_HID), jnp.float32),
    )(partials, partials, W, b.reshape(1, N_HID))


def kernel(x, batch, W, b):
    batch3 = batch.astype(jnp.int32).reshape(NW, N_CHUNKS, CHUNK)
    partials = _sc_segment_sum(x, batch3)
    return _tc_linear(partials, W, b)


# speedup vs baseline: 1.0419x; 1.0167x over previous
import functools

import jax
import jax.numpy as jnp
from jax import lax
from jax.experimental import pallas as pl
from jax.experimental.pallas import tpu as pltpu
from jax.experimental.pallas import tpu_sc as plsc

N_ROWS = 320000
N_SEG = 10000
N_HID = 128

NC = 2
NS = 16
NW = NC * NS
ROWS_PER_W = N_ROWS // NW
CHUNK = 80
N_CHUNKS = ROWS_PER_W // CHUNK
NBUF = 4
FULL_ROUNDS = 30
SEG_PER_TILE = 640
N_STAGE = 8


def _sc_segment_sum(x, batch3):
    mesh = plsc.VectorSubcoreMesh(
        core_axis_name="c", subcore_axis_name="s",
        num_cores=NC, num_subcores=NS)

    @functools.partial(
        pl.kernel,
        out_type=jax.ShapeDtypeStruct((NC * N_SEG, N_HID), jnp.float32),
        mesh=mesh,
        compiler_params=pltpu.CompilerParams(use_tc_tiling_on_sc=False),
        scratch_types=dict(
            pooled=pltpu.VMEM_SHARED((N_SEG, N_HID), jnp.float32),
            idx_all=pltpu.VMEM((N_CHUNKS, CHUNK), jnp.int32),
            rows=pltpu.VMEM((NBUF, CHUNK, N_HID), jnp.float32),
            isem=pltpu.SemaphoreType.DMA,
            lsem=pltpu.SemaphoreType.DMA((NBUF,)),
            ssem=pltpu.SemaphoreType.DMA((NBUF,)),
        ),
    )
    def body(x_hbm, batch3_hbm, out_hbm, *, pooled, idx_all, rows,
             isem, lsem, ssem):
        c = lax.axis_index("c")
        s = lax.axis_index("s")
        wid = c * NS + s
        row0 = wid * ROWS_PER_W

        def load_desc(i, b):
            return pltpu.make_async_copy(
                x_hbm.at[pl.ds(row0 + i * CHUNK, CHUNK)], rows.at[b],
                lsem.at[b])

        def scat_desc(i, b):
            return pltpu.make_async_copy(
                rows.at[b], pooled.at[idx_all.at[i]], ssem.at[b])

        idx_desc = pltpu.make_async_copy(batch3_hbm.at[wid], idx_all, isem)
        idx_desc.start()
        for b in range(1, NBUF):
            load_desc(b, b).start()

        zvec = jnp.zeros((16,), jnp.float32)

        def zrow(i, _):
            for j in range(N_HID // 16):
                rows[0, i, pl.ds(j * 16, 16)] = zvec
            return 0

        lax.fori_loop(0, CHUNK, zrow, 0)
        nseg_t = jnp.where(s == NS - 1,
                           N_SEG - (NS - 1) * SEG_PER_TILE, SEG_PER_TILE)
        for k in range(N_STAGE):
            seg0 = s * SEG_PER_TILE + k * CHUNK

            @pl.when(k * CHUNK < nseg_t)
            def _():
                pltpu.sync_copy(rows.at[0], pooled.at[pl.ds(seg0, CHUNK)])

        load_desc(0, 0).start()
        idx_desc.wait()
        plsc.subcore_barrier()

        def full_round(g, _):
            for b in range(NBUF):
                i = g * NBUF + b
                load_desc(i, b).wait()
                scat_desc(i, b).start(add=True)
            for b in range(NBUF):
                i = g * NBUF + b
                scat_desc(i, b).wait()

                @pl.when(i + NBUF < N_CHUNKS)
                def _():
                    load_desc(i + NBUF, b).start()
            return 0

        lax.fori_loop(0, FULL_ROUNDS + 1, full_round, 0)

        for b in range(N_CHUNKS - (FULL_ROUNDS + 1) * NBUF):
            i = (FULL_ROUNDS + 1) * NBUF + b
            load_desc(i, b).wait()
            scat_desc(i, b).start(add=True)
            scat_desc(i, b).wait()
        plsc.subcore_barrier()

        def rd_desc(k, p):
            seg0 = s * SEG_PER_TILE + k * CHUNK
            return pltpu.make_async_copy(
                pooled.at[pl.ds(seg0, CHUNK)], rows.at[p], lsem.at[p])

        def wr_desc(k, p):
            seg0 = s * SEG_PER_TILE + k * CHUNK
            return pltpu.make_async_copy(
                rows.at[p], out_hbm.at[pl.ds(c * N_SEG + seg0, CHUNK)],
                ssem.at[p])

        def drain_pair(m, _):
            for p in range(2):
                k = 2 * m + p

                @pl.when(k * CHUNK < nseg_t)
                def _():
                    @pl.when(m >= 1)
                    def _():
                        wr_desc(k - 2, p).wait()
                    rd_desc(k, p).start()
                    rd_desc(k, p).wait()
                    wr_desc(k, p).start()
            return 0

        lax.fori_loop(0, N_STAGE // 2, drain_pair, 0)
        def drain_wait(m, _):
            for p in range(2):
                k = 2 * m + p
                cond = jnp.logical_and(k * CHUNK < nseg_t,
                                       (k + 2) * CHUNK >= nseg_t)

                @pl.when(cond)
                def _():
                    wr_desc(k, p).wait()
            return 0

        lax.fori_loop(0, N_STAGE // 2, drain_wait, 0)

    return body(x, batch3)


ROW_BLK = 5000


def _mm_body(p0_ref, p1_ref, w_ref, b_ref, o_ref):
    pooled = p0_ref[...] + p1_ref[...]
    o_ref[...] = lax.dot_general(
        pooled, w_ref[...], (((1,), (1,)), ((), ())),
        preferred_element_type=jnp.float32) + b_ref[...]


def _tc_linear(partials, W, b):
    grid = N_SEG // ROW_BLK
    return pl.pallas_call(
        _mm_body,
        grid=(grid,),
        in_specs=[
            pl.BlockSpec((ROW_BLK, N_HID), lambda i: (i, 0)),
            pl.BlockSpec((ROW_BLK, N_HID), lambda i, g=grid: (i + g, 0)),
            pl.BlockSpec((N_HID, N_HID), lambda i: (0, 0)),
            pl.BlockSpec((1, N_HID), lambda i: (0, 0)),
        ],
        out_specs=pl.BlockSpec((ROW_BLK, N_HID), lambda i: (i, 0)),
        out_shape=jax.ShapeDtypeStruct((N_SEG, N_HID), jnp.float32),
    )(partials, partials, W, b.reshape(1, N_HID))


def kernel(x, batch, W, b):
    batch3 = batch.astype(jnp.int32).reshape(NW, N_CHUNKS, CHUNK)
    partials = _sc_segment_sum(x, batch3)
    return _tc_linear(partials, W, b)
